# Initial kernel scaffold; baseline (speedup 1.0000x reference)
import jax
import jax.numpy as jnp
from jax.experimental import pallas as pl


def kernel(seg_label, sentence_embedding, len_paper_list, Ws, bs):
    raise NotImplementedError("write your pallas kernel here")


# speedup vs baseline: 5.4597x; 5.4597x over previous
import jax
import jax.numpy as jnp
from jax import lax
from jax.experimental import pallas as pl
from jax.experimental.pallas import tpu as pltpu
from jax.experimental.pallas import tpu_sc as plsc

B, L, D, OUT = 8, 2048, 400, 1024
DP = 512
DQ = DP // 4
NV = DQ // 16
CH = 128
NCH = L // CH
SG = CH + 8
LP = L + 8
LT = 256
OT = OUT
EG = 64
LQ = L // 4


def _pool_body(emb, starts, lasts, mean_o, max_o, last_o, ns_o,
               xbuf, stbuf, labuf, sg_m, sg_x, sg_l, sg_n):
    c = lax.axis_index("c")
    s = lax.axis_index("s")
    wid = s * 2 + c
    p = wid // 4
    q = wid % 4
    doff = q * DQ
    pbase = p * L

    def chunk(ci, carry):
        done, aold = carry[1], carry[2]
        abase = pl.multiple_of((done // 8) * 8, 8)
        sh = abase - aold
        for r in range(8):
            for k in range(NV):
                vm = sg_m[sh + r, pl.ds(k * 16, 16)]
                vx = sg_x[sh + r, pl.ds(k * 16, 16)]
                vl = sg_l[sh + r, pl.ds(k * 16, 16)]
                sg_m[r, pl.ds(k * 16, 16)] = vm
                sg_x[r, pl.ds(k * 16, 16)] = vx
                sg_l[r, pl.ds(k * 16, 16)] = vl
        pltpu.sync_copy(emb.at[p, pl.ds(ci * CH, CH), pl.ds(doff, DQ)], xbuf)
        pltpu.sync_copy(starts.at[pl.ds(pbase + ci * CH, CH)],
                        stbuf.at[pl.ds(0, CH)])
        pltpu.sync_copy(lasts.at[pl.ds(pbase + ci * CH, CH)],
                        labuf.at[pl.ds(0, CH)])

        def sent(i, sc):
            cur, done, cnt = sc[0], sc[1], sc[3]
            rs = sc[4:4 + NV]
            rm = sc[4 + NV:4 + 2 * NV]
            st = stbuf[pl.ds(i, 16)][0] != 0
            cur = cur + jnp.where(st, 1, 0)
            x = [xbuf[i, pl.ds(k * 16, 16)] for k in range(NV)]
            rs = [jnp.where(st, x[k], rs[k] + x[k]) for k in range(NV)]
            rm = [jnp.where(st, x[k], jnp.maximum(rm[k], x[k]))
                  for k in range(NV)]
            cnt = jnp.where(st, 1.0, cnt + 1.0)
            lastf = labuf[pl.ds(i, 16)][0] != 0

            @pl.when(lastf)
            def _():
                row = cur - abase
                cv = jnp.full((16,), cnt, dtype=jnp.float32)
                for k in range(NV):
                    sg_m[row, pl.ds(k * 16, 16)] = rs[k] / cv
                    sg_x[row, pl.ds(k * 16, 16)] = rm[k]
                    sg_l[row, pl.ds(k * 16, 16)] = x[k]

            done = jnp.where(lastf, cur + 1, done)
            return (cur, done, sc[2], cnt) + tuple(rs) + tuple(rm)

        carry = lax.fori_loop(0, CH, sent, carry)
        pltpu.sync_copy(sg_m, mean_o.at[p, pl.ds(abase, SG), pl.ds(doff, DQ)])
        pltpu.sync_copy(sg_x, max_o.at[p, pl.ds(abase, SG), pl.ds(doff, DQ)])
        pltpu.sync_copy(sg_l, last_o.at[p, pl.ds(abase, SG), pl.ds(doff, DQ)])
        return carry[:2] + (abase,) + carry[3:]

    zero = jnp.zeros((16,), jnp.float32)
    init = (jnp.int32(-1), jnp.int32(0), jnp.int32(0),
            jnp.float32(0.0)) + (zero,) * (2 * NV)
    carry = lax.fori_loop(0, NCH, chunk, init)

    @pl.when(q == 0)
    def _():
        sg_n[...] = jnp.full((16,), carry[1], dtype=jnp.int32)
        pltpu.sync_copy(sg_n, ns_o.at[pl.ds(p * 16, 16)])


def _pool(emb_pad, starts, lasts):
    mesh = plsc.VectorSubcoreMesh(core_axis_name="c", subcore_axis_name="s")
    f32, i32 = jnp.float32, jnp.int32
    kern = pl.kernel(
        _pool_body,
        out_type=(
            jax.ShapeDtypeStruct((B, LP, DP), f32),
            jax.ShapeDtypeStruct((B, LP, DP), f32),
            jax.ShapeDtypeStruct((B, LP, DP), f32),
            jax.ShapeDtypeStruct((B * 16,), i32),
        ),
        mesh=mesh,
        scratch_types=[
            pltpu.VMEM((CH, DQ), f32),
            pltpu.VMEM((CH + 16,), i32),
            pltpu.VMEM((CH + 16,), i32),
            pltpu.VMEM((SG, DQ), f32),
            pltpu.VMEM((SG, DQ), f32),
            pltpu.VMEM((SG, DQ), f32),
            pltpu.VMEM((16,), i32),
        ],
    )
    return kern(emb_pad, starts, lasts)


def _mm_body(ns_ref, m_ref, x_ref, l_ref, wm_ref, wx_ref, wl_ref, b_ref,
             t_ref, lab_ref):
    b = pl.program_id(0)
    lt = pl.program_id(1)

    @pl.when(lt * LT < ns_ref[b * 16])
    def _():
        acc = jnp.dot(m_ref[0], wm_ref[...], preferred_element_type=jnp.float32)
        acc += jnp.dot(x_ref[0], wx_ref[...], preferred_element_type=jnp.float32)
        acc += jnp.dot(l_ref[0], wl_ref[...], preferred_element_type=jnp.float32)
        acc += b_ref[0:1, :]
        t_ref[...] = acc
        mx = jnp.max(acc, axis=-1, keepdims=True)
        iot = lax.broadcasted_iota(jnp.int32, (LT, OT), 1)
        cand = jnp.where(acc == mx, iot, OT)
        lab_ref[...] = jnp.min(cand, axis=-1).reshape(1, 1, LT)


def _matmul(n_seg, mean_o, max_o, last_o, wm, wx, wl, bias):
    f32, i32 = jnp.float32, jnp.int32
    nl = L // LT
    grid = (B, nl)
    xspec = pl.BlockSpec((1, LT, DP), lambda b, lt: (b, lt, 0))
    wspec = pl.BlockSpec((DP, OT), lambda b, lt: (0, 0))
    return pl.pallas_call(
        _mm_body,
        grid=grid,
        in_specs=[
            pl.BlockSpec(memory_space=pltpu.SMEM),
            xspec, xspec, xspec, wspec, wspec, wspec,
            pl.BlockSpec((8, OT), lambda b, lt: (0, 0)),
        ],
        out_specs=[
            pl.BlockSpec((LT, OT), lambda b, lt: (b * nl + lt, 0)),
            pl.BlockSpec((1, 1, LT), lambda b, lt: (b * nl + lt, 0, 0)),
        ],
        out_shape=[
            jax.ShapeDtypeStruct((B * L, OUT), f32),
            jax.ShapeDtypeStruct((B * nl, 1, LT), i32),
        ],
    )(n_seg, mean_o, max_o, last_o, wm, wx, wl, bias)


def _exp_body(topic, labseg, starts, topics_o, labels_o,
              rows_v, lbuf, stv, sidb, obuf, sem):
    c = lax.axis_index("c")
    s = lax.axis_index("s")
    wid = s * 2 + c
    p = wid // 4
    r = wid % 4
    row0 = r * LQ
    pbase = p * L

    pltpu.sync_copy(starts.at[pl.ds(pbase, L)], stv)
    pltpu.sync_copy(labseg.at[pl.ds(pbase, L)], lbuf)

    off = p * L - 1
    for g in range(L // 16):
        v = stv[pl.ds(g * 16, 16)]
        cs = plsc.cumsum(v)
        sidb[pl.ds(g * 16, 16)] = cs + off
        off = off + cs[15]

    for j in range(LQ // EG):
        for m in range(EG // 16):
            loc = sidb[pl.ds(row0 + j * EG + m * 16, 16)] - pbase
            obuf[pl.ds(j * EG + m * 16, 16)] = plsc.load_gather(lbuf, [loc])
    pltpu.sync_copy(obuf, labels_o.at[pl.ds(pbase + row0, LQ)])

    for j in range(LQ // EG):
        idx = sidb.at[pl.ds(row0 + j * EG, EG)]
        pltpu.async_copy(topic.at[idx], rows_v, sem).wait()
        pltpu.sync_copy(rows_v, topics_o.at[p, pl.ds(row0 + j * EG, EG), :])


def _expand(topic_res, labels_seg, starts):
    mesh = plsc.VectorSubcoreMesh(core_axis_name="c", subcore_axis_name="s")
    f32, i32 = jnp.float32, jnp.int32
    kern = pl.kernel(
        _exp_body,
        out_type=(
            jax.ShapeDtypeStruct((B, L, OUT), f32),
            jax.ShapeDtypeStruct((B * L,), i32),
        ),
        mesh=mesh,
        scratch_types=[
            pltpu.VMEM((EG, OUT), f32),
            pltpu.VMEM((L,), i32),
            pltpu.VMEM((L,), i32),
            pltpu.VMEM((L,), i32),
            pltpu.VMEM((LQ,), i32),
            pltpu.SemaphoreType.DMA,
        ],
        compiler_params=pltpu.CompilerParams(needs_layout_passes=False),
    )
    return kern(topic_res, labels_seg, starts)


def kernel(seg_label, sentence_embedding, len_paper_list, Ws, bs):
    f32 = jnp.float32
    emb_pad = jnp.pad(sentence_embedding, ((0, 0), (0, 0), (0, DP - D)))
    starts = seg_label.at[:, 0].set(1).reshape(B * L)
    lasts = jnp.concatenate(
        [seg_label[:, 1:], jnp.ones((B, 1), seg_label.dtype)],
        axis=1).reshape(B * L)
    wpad = jnp.zeros((DP - D, OUT), f32)
    wm = jnp.concatenate([Ws[0:D], wpad], axis=0)
    wx = jnp.concatenate([Ws[D:2 * D], wpad], axis=0)
    wl = jnp.concatenate([Ws[2 * D:3 * D], wpad], axis=0)
    bias = jnp.broadcast_to(bs[None, :], (8, OUT))

    mean_o, max_o, last_o, n_seg = _pool(emb_pad, starts, lasts)
    topic_res, lab3 = _matmul(n_seg, mean_o, max_o, last_o, wm, wx, wl, bias)
    labels_seg = lab3.reshape(B * L)
    topics, labels = _expand(topic_res, labels_seg, starts)
    return topics, labels.reshape(B, L)


# speedup vs baseline: 6.5636x; 1.2022x over previous
import jax
import jax.numpy as jnp
from jax import lax
from jax.experimental import pallas as pl
from jax.experimental.pallas import tpu as pltpu
from jax.experimental.pallas import tpu_sc as plsc

B, L, D, OUT = 8, 2048, 400, 1024
DP = 512
DQ = DP // 4
NV = DQ // 16
CH = 64
NCH = L // CH
SG = CH + 8
LP = L + 8
LT = 256
OT = OUT
EG = 64
LQ = L // 4


def _pool_body(emb, starts, lasts, mean_o, max_o, last_o, ns_o,
               xb0, xb1, st0, st1, lb0, lb1,
               m0, m1, x0, x1, l0, l1, sg_n, sem_in, sem_out):
    c = lax.axis_index("c")
    s = lax.axis_index("s")
    wid = s * 2 + c
    p = wid // 4
    q = wid % 4
    doff = q * DQ
    pbase = p * L
    xbufs, stbufs, labufs = (xb0, xb1), (st0, st1), (lb0, lb1)
    sgm, sgx, sgl = (m0, m1), (x0, x1), (l0, l1)

    def fire_in(ci, pr):
        pltpu.make_async_copy(
            emb.at[p, pl.ds(ci * CH, CH), pl.ds(doff, DQ)],
            xbufs[pr], sem_in).start()
        pltpu.make_async_copy(
            starts.at[pl.ds(pbase + ci * CH, CH)],
            stbufs[pr].at[pl.ds(0, CH)], sem_in).start()
        pltpu.make_async_copy(
            lasts.at[pl.ds(pbase + ci * CH, CH)],
            labufs[pr].at[pl.ds(0, CH)], sem_in).start()

    def wait_in(pr):
        pltpu.make_async_copy(
            emb.at[p, pl.ds(0, CH), pl.ds(doff, DQ)], xbufs[pr],
            sem_in).wait()
        pltpu.make_async_copy(
            starts.at[pl.ds(pbase, CH)], stbufs[pr].at[pl.ds(0, CH)],
            sem_in).wait()
        pltpu.make_async_copy(
            lasts.at[pl.ds(pbase, CH)], labufs[pr].at[pl.ds(0, CH)],
            sem_in).wait()

    def out_copies(pr, abase):
        return (
            pltpu.make_async_copy(
                sgm[pr], mean_o.at[p, pl.ds(abase, SG), pl.ds(doff, DQ)],
                sem_out),
            pltpu.make_async_copy(
                sgx[pr], max_o.at[p, pl.ds(abase, SG), pl.ds(doff, DQ)],
                sem_out),
            pltpu.make_async_copy(
                sgl[pr], last_o.at[p, pl.ds(abase, SG), pl.ds(doff, DQ)],
                sem_out),
        )

    fire_in(0, 0)

    def sup(S, carry):
        for b2 in range(2):
            ci = 2 * S + b2
            pr = b2
            wait_in(pr)
            if b2 == 0:
                fire_in(ci + 1, 1 - pr)
            else:
                @pl.when(S < NCH // 2 - 1)
                def _():
                    fire_in(ci + 1, 1 - pr)

            done, aold = carry[1], carry[2]
            abase = pl.multiple_of((done // 8) * 8, 8)
            sh = abase - aold
            for r in range(8):
                for k in range(NV):
                    vm = sgm[1 - pr][sh + r, pl.ds(k * 16, 16)]
                    vx = sgx[1 - pr][sh + r, pl.ds(k * 16, 16)]
                    vl = sgl[1 - pr][sh + r, pl.ds(k * 16, 16)]
                    sgm[pr][r, pl.ds(k * 16, 16)] = vm
                    sgx[pr][r, pl.ds(k * 16, 16)] = vx
                    sgl[pr][r, pl.ds(k * 16, 16)] = vl

            def sent(i, sc):
                cur, done, cnt = sc[0], sc[1], sc[3]
                rs = sc[4:4 + NV]
                rm = sc[4 + NV:4 + 2 * NV]
                st = stbufs[pr][pl.ds(i, 16)][0] != 0
                cur = cur + jnp.where(st, 1, 0)
                x = [xbufs[pr][i, pl.ds(k * 16, 16)] for k in range(NV)]
                rs = [jnp.where(st, x[k], rs[k] + x[k]) for k in range(NV)]
                rm = [jnp.where(st, x[k], jnp.maximum(rm[k], x[k]))
                      for k in range(NV)]
                cnt = jnp.where(st, 1.0, cnt + 1.0)
                lastf = labufs[pr][pl.ds(i, 16)][0] != 0

                @pl.when(lastf)
                def _():
                    row = cur - abase
                    cv = jnp.full((16,), cnt, dtype=jnp.float32)
                    for k in range(NV):
                        sgm[pr][row, pl.ds(k * 16, 16)] = rs[k] / cv
                        sgx[pr][row, pl.ds(k * 16, 16)] = rm[k]
                        sgl[pr][row, pl.ds(k * 16, 16)] = x[k]

                done = jnp.where(lastf, cur + 1, done)
                return (cur, done, sc[2], cnt) + tuple(rs) + tuple(rm)

            carry = lax.fori_loop(0, CH, sent, carry)
            if b2 == 0:
                @pl.when(S >= 1)
                def _():
                    for cp in out_copies(1 - pr, 0):
                        cp.wait()
            else:
                for cp in out_copies(1 - pr, 0):
                    cp.wait()
            for cp in out_copies(pr, abase):
                cp.start()
            carry = carry[:2] + (abase,) + carry[3:]
        return carry

    zero = jnp.zeros((16,), jnp.float32)
    init = (jnp.int32(-1), jnp.int32(0), jnp.int32(0),
            jnp.float32(0.0)) + (zero,) * (2 * NV)
    carry = lax.fori_loop(0, NCH // 2, sup, init)
    for cp in out_copies(1, 0):
        cp.wait()

    @pl.when(q == 0)
    def _():
        sg_n[...] = jnp.full((16,), carry[1], dtype=jnp.int32)
        pltpu.sync_copy(sg_n, ns_o.at[pl.ds(p * 16, 16)])


def _pool(emb_pad, starts, lasts):
    mesh = plsc.VectorSubcoreMesh(core_axis_name="c", subcore_axis_name="s")
    f32, i32 = jnp.float32, jnp.int32
    kern = pl.kernel(
        _pool_body,
        out_type=(
            jax.ShapeDtypeStruct((B, LP, DP), f32),
            jax.ShapeDtypeStruct((B, LP, DP), f32),
            jax.ShapeDtypeStruct((B, LP, DP), f32),
            jax.ShapeDtypeStruct((B * 16,), i32),
        ),
        mesh=mesh,
        scratch_types=[
            pltpu.VMEM((CH, DQ), f32),
            pltpu.VMEM((CH, DQ), f32),
            pltpu.VMEM((CH + 16,), i32),
            pltpu.VMEM((CH + 16,), i32),
            pltpu.VMEM((CH + 16,), i32),
            pltpu.VMEM((CH + 16,), i32),
            pltpu.VMEM((SG, DQ), f32),
            pltpu.VMEM((SG, DQ), f32),
            pltpu.VMEM((SG, DQ), f32),
            pltpu.VMEM((SG, DQ), f32),
            pltpu.VMEM((SG, DQ), f32),
            pltpu.VMEM((SG, DQ), f32),
            pltpu.VMEM((16,), i32),
            pltpu.SemaphoreType.DMA,
            pltpu.SemaphoreType.DMA,
        ],
    )
    return kern(emb_pad, starts, lasts)


def _mm_body(ns_ref, m_ref, x_ref, l_ref, wm_ref, wx_ref, wl_ref, b_ref,
             t_ref, lab_ref):
    b = pl.program_id(0)
    lt = pl.program_id(1)

    @pl.when(lt * LT < ns_ref[b * 16])
    def _():
        acc = jnp.dot(m_ref[0], wm_ref[...], preferred_element_type=jnp.float32)
        acc += jnp.dot(x_ref[0], wx_ref[...], preferred_element_type=jnp.float32)
        acc += jnp.dot(l_ref[0], wl_ref[...], preferred_element_type=jnp.float32)
        acc += b_ref[0:1, :]
        t_ref[...] = acc
        mx = jnp.max(acc, axis=-1, keepdims=True)
        iot = lax.broadcasted_iota(jnp.int32, (LT, OT), 1)
        cand = jnp.where(acc == mx, iot, OT)
        lab_ref[...] = jnp.min(cand, axis=-1).reshape(1, 1, LT)


def _matmul(n_seg, mean_o, max_o, last_o, wm, wx, wl, bias):
    f32, i32 = jnp.float32, jnp.int32
    nl = L // LT
    grid = (B, nl)
    xspec = pl.BlockSpec((1, LT, DP), lambda b, lt: (b, lt, 0))
    wspec = pl.BlockSpec((DP, OT), lambda b, lt: (0, 0))
    return pl.pallas_call(
        _mm_body,
        grid=grid,
        in_specs=[
            pl.BlockSpec(memory_space=pltpu.SMEM),
            xspec, xspec, xspec, wspec, wspec, wspec,
            pl.BlockSpec((8, OT), lambda b, lt: (0, 0)),
        ],
        out_specs=[
            pl.BlockSpec((LT, OT), lambda b, lt: (b * nl + lt, 0)),
            pl.BlockSpec((1, 1, LT), lambda b, lt: (b * nl + lt, 0, 0)),
        ],
        out_shape=[
            jax.ShapeDtypeStruct((B * L, OUT), f32),
            jax.ShapeDtypeStruct((B * nl, 1, LT), i32),
        ],
    )(n_seg, mean_o, max_o, last_o, wm, wx, wl, bias)


def _exp_body(topic, labseg, starts, topics_o, labels_o,
              rows_v, lbuf, stv, sidb, obuf, sem):
    c = lax.axis_index("c")
    s = lax.axis_index("s")
    wid = s * 2 + c
    p = wid // 4
    r = wid % 4
    row0 = r * LQ
    pbase = p * L

    pltpu.sync_copy(starts.at[pl.ds(pbase, L)], stv)
    pltpu.sync_copy(labseg.at[pl.ds(pbase, L)], lbuf)

    off = p * L - 1
    for g in range(L // 16):
        v = stv[pl.ds(g * 16, 16)]
        cs = plsc.cumsum(v)
        sidb[pl.ds(g * 16, 16)] = cs + off
        off = off + cs[15]

    for j in range(LQ // EG):
        for m in range(EG // 16):
            loc = sidb[pl.ds(row0 + j * EG + m * 16, 16)] - pbase
            obuf[pl.ds(j * EG + m * 16, 16)] = plsc.load_gather(lbuf, [loc])
    pltpu.sync_copy(obuf, labels_o.at[pl.ds(pbase + row0, LQ)])

    for j in range(LQ // EG):
        idx = sidb.at[pl.ds(row0 + j * EG, EG)]
        pltpu.async_copy(topic.at[idx], rows_v, sem).wait()
        pltpu.sync_copy(rows_v, topics_o.at[p, pl.ds(row0 + j * EG, EG), :])


def _expand(topic_res, labels_seg, starts):
    mesh = plsc.VectorSubcoreMesh(core_axis_name="c", subcore_axis_name="s")
    f32, i32 = jnp.float32, jnp.int32
    kern = pl.kernel(
        _exp_body,
        out_type=(
            jax.ShapeDtypeStruct((B, L, OUT), f32),
            jax.ShapeDtypeStruct((B * L,), i32),
        ),
        mesh=mesh,
        scratch_types=[
            pltpu.VMEM((EG, OUT), f32),
            pltpu.VMEM((L,), i32),
            pltpu.VMEM((L,), i32),
            pltpu.VMEM((L,), i32),
            pltpu.VMEM((LQ,), i32),
            pltpu.SemaphoreType.DMA,
        ],
        compiler_params=pltpu.CompilerParams(needs_layout_passes=False),
    )
    return kern(topic_res, labels_seg, starts)


def kernel(seg_label, sentence_embedding, len_paper_list, Ws, bs):
    f32 = jnp.float32
    emb_pad = jnp.pad(sentence_embedding, ((0, 0), (0, 0), (0, DP - D)))
    starts = seg_label.at[:, 0].set(1).reshape(B * L)
    lasts = jnp.concatenate(
        [seg_label[:, 1:], jnp.ones((B, 1), seg_label.dtype)],
        axis=1).reshape(B * L)
    wpad = jnp.zeros((DP - D, OUT), f32)
    wm = jnp.concatenate([Ws[0:D], wpad], axis=0)
    wx = jnp.concatenate([Ws[D:2 * D], wpad], axis=0)
    wl = jnp.concatenate([Ws[2 * D:3 * D], wpad], axis=0)
    bias = jnp.broadcast_to(bs[None, :], (8, OUT))

    mean_o, max_o, last_o, n_seg = _pool(emb_pad, starts, lasts)
    topic_res, lab3 = _matmul(n_seg, mean_o, max_o, last_o, wm, wx, wl, bias)
    labels_seg = lab3.reshape(B * L)
    topics, labels = _expand(topic_res, labels_seg, starts)
    return topics, labels.reshape(B, L)
